# baseline (speedup 1.0000x reference)
import jax
import jax.numpy as jnp
from jax.experimental import pallas as pl
from jax.experimental.pallas import tpu as pltpu


def _body(x_hbm, w_ref, b_ref, o_ref, xbuf, sems):
    half = xbuf.shape[1]
    c0 = pltpu.make_async_copy(x_hbm.at[pl.ds(0, half), :], xbuf.at[0],
                               sems.at[0])
    c0.start()
    c1 = pltpu.make_async_copy(x_hbm.at[pl.ds(half, half), :], xbuf.at[1],
                               sems.at[1])
    c1.start()
    w = w_ref[...]
    b = b_ref[...]
    dn = (((1,), (0,)), ((), ()))
    c0.wait()
    o_ref[pl.ds(0, half), :] = jax.lax.dot_general(
        jnp.maximum(xbuf[0], 0.0), w, dn,
        preferred_element_type=jnp.float32) + b
    c1.wait()
    o_ref[pl.ds(half, half), :] = jax.lax.dot_general(
        jnp.maximum(xbuf[1], 0.0), w, dn,
        preferred_element_type=jnp.float32) + b


def kernel(x_subject, x_region, edge_index_sr, edge_index_rr, edge_attr_sr,
           edge_attr_rr, sage_Wl0, sage_bl0, sage_Wr0, gcn_W0, gcn_b0,
           sage_Wl1, sage_bl1, sage_Wr1, gcn_W1, gcn_b1, lin_W, lin_b):
    m, d = x_subject.shape
    out_dim = lin_W.shape[1]
    return pl.pallas_call(
        _body,
        in_specs=[
            pl.BlockSpec(memory_space=pltpu.MemorySpace.HBM),
            pl.BlockSpec(memory_space=pltpu.MemorySpace.VMEM),
            pl.BlockSpec(memory_space=pltpu.MemorySpace.VMEM),
        ],
        out_specs=pl.BlockSpec(memory_space=pltpu.MemorySpace.VMEM),
        out_shape=jax.ShapeDtypeStruct((m, out_dim), jnp.float32),
        scratch_shapes=[
            pltpu.VMEM((2, m // 2, d), jnp.float32),
            pltpu.SemaphoreType.DMA((2,)),
        ],
    )(x_subject, lin_W, lin_b.reshape(1, out_dim))
